# baseline (speedup 1.0000x reference)
import functools

import jax
import jax.numpy as jnp
from jax import lax
from jax.experimental import pallas as pl
from jax.experimental.pallas import tpu as pltpu
from jax.experimental.pallas import tpu_sc as plsc

_PRE_SEQ_LEN = 128
_EMB = 18432
_B = 64 * 128
_TC_COLS = 12288
_SC_COLS = _EMB - _TC_COLS
_NC, _NS = 2, 16
_COLS = _SC_COLS // _NC
_RPW = _B // _NS
_K = 16
_NBUF = 2
_STEPS = _RPW // _K
_TROWS = _PRE_SEQ_LEN // _NS


def _sc_body(table_hbm, idx_hbm, out_hbm, table_s, idx_v, *sems):
    c = lax.axis_index("c")
    s = lax.axis_index("s")
    col0 = _TC_COLS + c * _COLS
    row0 = s * _RPW

    pltpu.sync_copy(
        table_hbm.at[pl.ds(s * _TROWS, _TROWS), pl.ds(col0, _COLS)],
        table_s.at[pl.ds(s * _TROWS, _TROWS)])
    pltpu.sync_copy(idx_hbm.at[pl.ds(row0, _RPW)], idx_v)
    plsc.subcore_barrier()

    def row_copy(row, t, buf):
        return pltpu.make_async_copy(
            table_s.at[pl.ds(t, 1)],
            out_hbm.at[pl.ds(row0 + row, 1), pl.ds(col0, _COLS)],
            sems[buf])

    def start_flight(g, buf):
        vec = idx_v[pl.ds(g * _K, _K)]
        for j in range(_K):
            row_copy(g * _K + j, vec[j], buf).start()

    def wait_flight(g, buf):
        vec = idx_v[pl.ds(g * _K, _K)]
        for j in range(_K):
            row_copy(g * _K + j, vec[j], buf).wait()

    for b in range(_NBUF):
        start_flight(b, b)

    def outer(i, _):
        base = i * _NBUF
        for b in range(_NBUF):
            wait_flight(base + b, b)
            start_flight(base + _NBUF + b, b)
        return 0

    lax.fori_loop(0, _STEPS // _NBUF - 1, outer, 0)

    last = _STEPS - _NBUF
    for b in range(_NBUF):
        wait_flight(last + b, b)


def _sc_stage(table, idx):
    mesh = plsc.VectorSubcoreMesh(core_axis_name="c", subcore_axis_name="s")
    f = pl.kernel(
        _sc_body,
        out_type=jax.ShapeDtypeStruct((_B, _EMB), jnp.float32),
        mesh=mesh,
        scratch_types=[
            pltpu.VMEM_SHARED((_PRE_SEQ_LEN, _COLS), jnp.float32),
            pltpu.VMEM((_RPW,), jnp.int32),
        ] + [pltpu.SemaphoreType.DMA] * _NBUF,
    )
    return f(table, idx)


_TC_BLK = 3072


def _tc_body(idx_ref, table_ref, buf_ref, out_ref):
    del buf_ref
    idx = idx_ref[0, 0, :]
    iota = lax.broadcasted_iota(jnp.int32, (_PRE_SEQ_LEN, _PRE_SEQ_LEN), 0)
    onehot = (idx[None, :] == iota).astype(jnp.float32)
    out_ref[0] = lax.dot_general(
        onehot, table_ref[...],
        dimension_numbers=(((0,), (0,)), ((), ())),
        preferred_element_type=jnp.float32)


def _tc_stage(idx3, table, buf):
    return pl.pallas_call(
        _tc_body,
        grid=(64, _TC_COLS // _TC_BLK),
        in_specs=[
            pl.BlockSpec((1, 1, _PRE_SEQ_LEN), lambda b, j: (b, 0, 0)),
            pl.BlockSpec((_PRE_SEQ_LEN, _TC_BLK), lambda b, j: (0, j)),
            pl.BlockSpec(memory_space=pl.ANY),
        ],
        out_specs=pl.BlockSpec((1, _PRE_SEQ_LEN, _TC_BLK),
                               lambda b, j: (b, 0, j)),
        out_shape=jax.ShapeDtypeStruct((64, _PRE_SEQ_LEN, _EMB), jnp.float32),
        input_output_aliases={2: 0},
    )(idx3, table, buf)


@jax.jit
def _gather(prefix, table):
    idx = prefix.reshape(_B)
    sc_out = _sc_stage(table, idx)
    buf = sc_out.reshape(64, _PRE_SEQ_LEN, _EMB)
    return _tc_stage(prefix.reshape(64, 1, _PRE_SEQ_LEN), table, buf)


def kernel(prefix, embedding):
    return _gather(prefix, embedding)


# speedup vs baseline: 1.8629x; 1.8629x over previous
import functools

import jax
import jax.numpy as jnp
from jax import lax
from jax.experimental import pallas as pl
from jax.experimental.pallas import tpu as pltpu
from jax.experimental.pallas import tpu_sc as plsc

_PRE_SEQ_LEN = 128
_EMB = 18432
_B = 64 * 128
_TC_COLS = 12288
_SC_COLS = _EMB - _TC_COLS
_NC, _NS = 2, 16
_COLS = _SC_COLS // _NC
_RPW = _B // _NS
_K = 16
_NBUF = 2
_STEPS = _RPW // _K
_TROWS = _PRE_SEQ_LEN // _NS


def _sc_body(table_hbm, idx_hbm, out_hbm, table_s, idx_v, *sems):
    c = lax.axis_index("c")
    s = lax.axis_index("s")
    col0 = _TC_COLS + c * _COLS
    row0 = s * _RPW

    pltpu.sync_copy(
        table_hbm.at[pl.ds(s * _TROWS, _TROWS), pl.ds(col0, _COLS)],
        table_s.at[pl.ds(s * _TROWS, _TROWS)])
    pltpu.sync_copy(idx_hbm.at[pl.ds(row0, _RPW)], idx_v)
    plsc.subcore_barrier()

    def row_copy(row, t, buf):
        return pltpu.make_async_copy(
            table_s.at[pl.ds(t, 1)],
            out_hbm.at[pl.ds(row0 + row, 1), pl.ds(col0, _COLS)],
            sems[buf])

    def start_flight(g, buf):
        vec = idx_v[pl.ds(g * _K, _K)]
        for j in range(_K):
            row_copy(g * _K + j, vec[j], buf).start()

    def wait_flight(g, buf):
        vec = idx_v[pl.ds(g * _K, _K)]
        for j in range(_K):
            row_copy(g * _K + j, vec[j], buf).wait()

    for b in range(_NBUF):
        start_flight(b, b)

    def outer(i, _):
        base = i * _NBUF
        for b in range(_NBUF):
            wait_flight(base + b, b)
            start_flight(base + _NBUF + b, b)
        return 0

    lax.fori_loop(0, _STEPS // _NBUF - 1, outer, 0)

    last = _STEPS - _NBUF
    for b in range(_NBUF):
        wait_flight(last + b, b)


def _sc_stage(table, idx):
    mesh = plsc.VectorSubcoreMesh(core_axis_name="c", subcore_axis_name="s")
    f = pl.kernel(
        _sc_body,
        out_type=jax.ShapeDtypeStruct((_B, _EMB), jnp.float32),
        mesh=mesh,
        scratch_types=[
            pltpu.VMEM_SHARED((_PRE_SEQ_LEN, _COLS), jnp.float32),
            pltpu.VMEM((_RPW,), jnp.int32),
        ] + [pltpu.SemaphoreType.DMA] * _NBUF,
    )
    return f(table, idx)


def _tc_body(idx_ref, table_ref, buf_ref, out_ref):
    del buf_ref
    idx = idx_ref[0, 0, :]
    iota = lax.broadcasted_iota(jnp.int32, (_PRE_SEQ_LEN, _PRE_SEQ_LEN), 0)
    onehot = (idx[None, :] == iota).astype(jnp.float32)
    out_ref[0] = lax.dot_general(
        onehot, table_ref[...],
        dimension_numbers=(((0,), (0,)), ((), ())),
        preferred_element_type=jnp.float32)


def _tc_stage(idx3, table, buf):
    return pl.pallas_call(
        _tc_body,
        grid=(64,),
        in_specs=[
            pl.BlockSpec((1, 1, _PRE_SEQ_LEN), lambda b: (b, 0, 0)),
            pl.BlockSpec((_PRE_SEQ_LEN, _TC_COLS), lambda b: (0, 0)),
            pl.BlockSpec(memory_space=pl.ANY),
        ],
        out_specs=pl.BlockSpec((1, _PRE_SEQ_LEN, _TC_COLS),
                               lambda b: (b, 0, 0)),
        out_shape=jax.ShapeDtypeStruct((64, _PRE_SEQ_LEN, _EMB), jnp.float32),
        input_output_aliases={2: 0},
    )(idx3, table, buf)


@jax.jit
def _gather(prefix, table):
    idx = prefix.reshape(_B)
    sc_out = _sc_stage(table, idx)
    buf = sc_out.reshape(64, _PRE_SEQ_LEN, _EMB)
    return _tc_stage(prefix.reshape(64, 1, _PRE_SEQ_LEN), table, buf)


def kernel(prefix, embedding):
    return _gather(prefix, embedding)
